# baseline (speedup 1.0000x reference)
import functools

import jax
import jax.numpy as jnp
from jax import lax
from jax.experimental import pallas as pl
from jax.experimental.pallas import tpu as pltpu
from jax.experimental.pallas import tpu_sc as plsc

F32 = jnp.float32


def _deg_kernel_body(NP, EA, dst_hbm, hist_out, idx_v, hist_v):
    c = lax.axis_index("c")
    s = lax.axis_index("s")
    wid = s * 2 + c
    z16 = jnp.zeros((16,), F32)

    def zero_body(i, carry):
        hist_v[pl.ds(i * 16, 16)] = z16
        return carry

    lax.fori_loop(0, NP // 16, zero_body, 0)
    pltpu.sync_copy(dst_hbm.at[pl.ds(wid * EA, EA)], idx_v)
    ones16 = jnp.ones((16,), F32)
    UN = 5

    def hist_body(j, carry):
        for k in range(UN):
            idx = idx_v[pl.ds(j * (16 * UN) + k * 16, 16)]
            plsc.addupdate_scatter(hist_v, [idx], ones16)
        return carry

    lax.fori_loop(0, EA // (16 * UN), hist_body, 0)
    pltpu.sync_copy(hist_v, hist_out.at[wid])


def _agg_kernel_body(N, NP, E, EPT, H, NB, y2_hbm, src_hbm, dst_hbm, agg_out,
                     src_v, dst_v, rows, acc_sp, gsems, ssems):
    c = lax.axis_index("c")
    s = lax.axis_index("s")
    NPT = NP // 16
    ER = E // 16
    z16 = jnp.zeros((16,), F32)
    zbuf = rows[0]

    def zbuf_body(i, carry):
        for k in range(H // 16):
            zbuf[i, pl.ds(k * 16, 16)] = z16
        return carry

    lax.fori_loop(0, 128, zbuf_body, 0)

    base = s * NPT
    off = 0
    while off < NPT:
        sz = min(128, NPT - off)
        pltpu.sync_copy(zbuf.at[pl.ds(0, sz)], acc_sp.at[pl.ds(base + off, sz), :])
        off += sz
    plsc.subcore_barrier()

    pltpu.sync_copy(src_hbm.at[pl.ds(s * ER, ER)], src_v.at[pl.ds(0, ER)])
    pltpu.sync_copy(dst_hbm.at[pl.ds(s * ER, ER)], dst_v.at[pl.ds(0, ER)])
    iota16 = lax.iota(jnp.int32, 16)
    for k in range((EPT - ER) // 16):
        padv = N + lax.rem(iota16 + (k * 16), NP - N)
        src_v[pl.ds(ER + k * 16, 16)] = padv
        dst_v[pl.ds(ER + k * 16, 16)] = padv

    nch = EPT // 128
    niter = nch // (2 * NB)
    s0 = tuple(range(NB))
    s1 = tuple(range(NB, 2 * NB))

    def run_edges(yref):
        def g_start(jc, b):
            pltpu.async_copy(yref.at[src_v.at[pl.ds(jc * 128, 128)]],
                             rows[b], gsems[b])

        def g_wait(b):
            pltpu.make_async_copy(yref.at[src_v.at[pl.ds(0, 128)]],
                                  rows[b], gsems[b]).wait()

        def s_start(jc, b):
            pltpu.async_copy(rows[b], acc_sp.at[dst_v.at[pl.ds(jc * 128, 128)]],
                             ssems[b], add=True)

        def s_wait(b):
            pltpu.make_async_copy(rows[b], acc_sp.at[dst_v.at[pl.ds(0, 128)]],
                                  ssems[b]).wait()

        for i, b in enumerate(s0):
            g_start(i, b)

        def body(j, carry):
            bse = 2 * j * NB
            for i, b in enumerate(s0):
                g_wait(b)
                s_start(bse + i, b)
            for i, b in enumerate(s1):
                @pl.when(j > 0)
                def _(b=b):
                    s_wait(b)
                g_start(bse + NB + i, b)
            for i, b in enumerate(s1):
                g_wait(b)
                s_start(bse + NB + i, b)
            for i, b in enumerate(s0):
                s_wait(b)

                @pl.when(j < niter - 1)
                def _(i=i, b=b):
                    g_start(bse + 2 * NB + i, b)
            return carry

        lax.fori_loop(0, niter, body, 0)
        for b in s1:
            s_wait(b)

    @pl.when(c == 0)
    def _():
        run_edges(y2_hbm.at[0])

    @pl.when(c == 1)
    def _():
        run_edges(y2_hbm.at[1])
    plsc.subcore_barrier()
    pltpu.sync_copy(acc_sp.at[pl.ds(base, NPT), :], agg_out.at[c, pl.ds(base, NPT), :])


def _dense_kernel_body(H, R, x_ref, hist_ref, wg_ref, y_ref, dis_ref):
    xb = x_ref[...]
    xw = jnp.dot(xb.astype(jnp.bfloat16), wg_ref[...].astype(jnp.bfloat16),
                 preferred_element_type=F32)
    for j in range(R // 128):
        deg = jnp.sum(hist_ref[:, j * 128:(j + 1) * 128], axis=0,
                      keepdims=True) + 1.0
        dis = lax.rsqrt(deg)
        dis_ref[j] = dis
        dis_col = jnp.transpose(dis)
        y = dis_col * xw[j * 128:(j + 1) * 128]
        y_ref[0, j * 128:(j + 1) * 128] = y[:, :H]
        y_ref[1, j * 128:(j + 1) * 128] = y[:, H:]


def _final_kernel_body(R, x_ref, agg_ref, dis_ref, wg_ref, wl_ref, bg_ref, bl_ref,
                       out_ref):
    xb = x_ref[...]
    agg = jnp.concatenate([agg_ref[0], agg_ref[1]], axis=1)
    xb16 = xb.astype(jnp.bfloat16)
    xw = jnp.dot(xb16, wg_ref[...].astype(jnp.bfloat16),
                 preferred_element_type=F32)
    skip = jnp.dot(xb16, wl_ref[...].astype(jnp.bfloat16),
                   preferred_element_type=F32)
    rows = []
    for j in range(R // 128):
        sl = slice(j * 128, (j + 1) * 128)
        dis_col = jnp.transpose(dis_ref[j])
        t = agg[sl] + dis_col * xw[sl]
        rows.append(dis_col * t + skip[sl])
    sv = jnp.concatenate(rows, axis=0) + bg_ref[...] + bl_ref[...]
    u = jnp.exp(jnp.minimum(sv, 20.0))
    w = (1.0 + u) * (1.0 + u)
    out_ref[...] = sv * ((w - 1.0) / (w + 1.0))


def kernel(x, edge_index, W_gcn, b_gcn, W_lin, b_lin):
    N, D = x.shape
    E = edge_index.shape[1]
    H = D // 2
    NP = ((N + 127) // 128) * 128
    EP = ((E + 16383) // 16384) * 16384
    if NP == N:
        NP += 128
    G = NP // 128
    EA = E // 32
    EPT = EP // 16
    NB = 2

    src_flat = edge_index[0]
    dst_flat = edge_index[1]
    bg2 = b_gcn.reshape(1, D)
    bl2 = b_lin.reshape(1, D)

    mesh = plsc.VectorSubcoreMesh(core_axis_name="c", subcore_axis_name="s")

    deg_kernel = functools.partial(
        pl.kernel,
        out_type=jax.ShapeDtypeStruct((32, NP), F32),
        mesh=mesh,
        compiler_params=pltpu.CompilerParams(needs_layout_passes=False),
        scratch_types=[
            pltpu.VMEM((EA,), jnp.int32),
            pltpu.VMEM((NP,), F32),
        ],
    )(functools.partial(_deg_kernel_body, NP, EA))
    hist32 = deg_kernel(dst_flat)

    R = 512
    y2, dis = pl.pallas_call(
        functools.partial(_dense_kernel_body, H, R),
        grid=((NP + R - 1) // R,),
        in_specs=[
            pl.BlockSpec((R, D), lambda i: (i, 0)),
            pl.BlockSpec((32, R), lambda i: (0, i)),
            pl.BlockSpec((D, D), lambda i: (0, 0)),
        ],
        out_specs=[
            pl.BlockSpec((2, R, H), lambda i: (0, i, 0)),
            pl.BlockSpec((R // 128, 1, 128), lambda i: (i, 0, 0)),
        ],
        out_shape=[
            jax.ShapeDtypeStruct((2, NP, H), F32),
            jax.ShapeDtypeStruct((G, 1, 128), F32),
        ],
    )(x, hist32, W_gcn)

    agg_kernel = functools.partial(
        pl.kernel,
        out_type=jax.ShapeDtypeStruct((2, NP, H), F32),
        mesh=mesh,
        compiler_params=pltpu.CompilerParams(
            needs_layout_passes=False, use_tc_tiling_on_sc=False),
        scratch_types=[
            pltpu.VMEM((EPT,), jnp.int32),
            pltpu.VMEM((EPT,), jnp.int32),
            [pltpu.VMEM((128, H), F32)] * (2 * NB),
            pltpu.VMEM_SHARED((NP, H), F32),
            [pltpu.SemaphoreType.DMA] * (2 * NB),
            [pltpu.SemaphoreType.DMA] * (2 * NB),
        ],
    )(functools.partial(_agg_kernel_body, N, NP, E, EPT, H, NB))
    agg2 = agg_kernel(y2, src_flat, dst_flat)

    out = pl.pallas_call(
        functools.partial(_final_kernel_body, R),
        grid=((N + R - 1) // R,),
        in_specs=[
            pl.BlockSpec((R, D), lambda i: (i, 0)),
            pl.BlockSpec((2, R, H), lambda i: (0, i, 0)),
            pl.BlockSpec((R // 128, 1, 128), lambda i: (i, 0, 0)),
            pl.BlockSpec((D, D), lambda i: (0, 0)),
            pl.BlockSpec((D, D), lambda i: (0, 0)),
            pl.BlockSpec((1, D), lambda i: (0, 0)),
            pl.BlockSpec((1, D), lambda i: (0, 0)),
        ],
        out_specs=pl.BlockSpec((R, D), lambda i: (i, 0)),
        out_shape=jax.ShapeDtypeStruct((N, D), F32),
    )(x, agg2, dis, W_gcn, W_lin, bg2, bl2)

    return out


# speedup vs baseline: 1.0552x; 1.0552x over previous
import functools

import jax
import jax.numpy as jnp
from jax import lax
from jax.experimental import pallas as pl
from jax.experimental.pallas import tpu as pltpu
from jax.experimental.pallas import tpu_sc as plsc

F32 = jnp.float32


def _deg_kernel_body(NP, EA, dst_hbm, hist_out, idx_v, hist_v):
    c = lax.axis_index("c")
    s = lax.axis_index("s")
    wid = s * 2 + c
    z16 = jnp.zeros((16,), F32)

    def zero_body(i, carry):
        hist_v[pl.ds(i * 16, 16)] = z16
        return carry

    lax.fori_loop(0, NP // 16, zero_body, 0)
    pltpu.sync_copy(dst_hbm.at[pl.ds(wid * EA, EA)], idx_v)
    ones16 = jnp.ones((16,), F32)
    UN = 5

    def hist_body(j, carry):
        for k in range(UN):
            idx = idx_v[pl.ds(j * (16 * UN) + k * 16, 16)]
            plsc.addupdate_scatter(hist_v, [idx], ones16)
        return carry

    lax.fori_loop(0, EA // (16 * UN), hist_body, 0)
    pltpu.sync_copy(hist_v, hist_out.at[wid])


def _agg_kernel_body(N, NP, E, EPT, H, NB, y2_hbm, src_hbm, dst_hbm, agg_out,
                     src_v, dst_v, rows, acc_sp, gsems, ssems):
    c = lax.axis_index("c")
    s = lax.axis_index("s")
    NPT = NP // 16
    ER = E // 16
    z16 = jnp.zeros((16,), F32)
    zbuf = rows[0]

    def zbuf_body(i, carry):
        for k in range(H // 16):
            zbuf[i, pl.ds(k * 16, 16)] = z16
        return carry

    lax.fori_loop(0, 128, zbuf_body, 0)

    base = s * NPT
    off = 0
    while off < NPT:
        sz = min(128, NPT - off)
        pltpu.sync_copy(zbuf.at[pl.ds(0, sz)], acc_sp.at[pl.ds(base + off, sz), :])
        off += sz
    plsc.subcore_barrier()

    pltpu.sync_copy(src_hbm.at[pl.ds(s * ER, ER)], src_v.at[pl.ds(0, ER)])
    pltpu.sync_copy(dst_hbm.at[pl.ds(s * ER, ER)], dst_v.at[pl.ds(0, ER)])
    iota16 = lax.iota(jnp.int32, 16)
    for k in range((EPT - ER) // 16):
        padv = N + lax.rem(iota16 + (k * 16), NP - N)
        src_v[pl.ds(ER + k * 16, 16)] = padv
        dst_v[pl.ds(ER + k * 16, 16)] = padv

    nch = EPT // 128
    niter = nch // (2 * NB)
    s0 = tuple(range(NB))
    s1 = tuple(range(NB, 2 * NB))

    def run_edges(yref):
        def g_start(jc, b):
            pltpu.async_copy(yref.at[src_v.at[pl.ds(jc * 128, 128)]],
                             rows[b], gsems[b])

        def g_wait(b):
            pltpu.make_async_copy(yref.at[src_v.at[pl.ds(0, 128)]],
                                  rows[b], gsems[b]).wait()

        def s_start(jc, b):
            pltpu.async_copy(rows[b], acc_sp.at[dst_v.at[pl.ds(jc * 128, 128)]],
                             ssems[b], add=True)

        def s_wait(b):
            pltpu.make_async_copy(rows[b], acc_sp.at[dst_v.at[pl.ds(0, 128)]],
                                  ssems[b]).wait()

        for i, b in enumerate(s0):
            g_start(i, b)

        def body(j, carry):
            bse = 2 * j * NB
            for i, b in enumerate(s0):
                g_wait(b)
                s_start(bse + i, b)
            for i, b in enumerate(s1):
                @pl.when(j > 0)
                def _(b=b):
                    s_wait(b)
                g_start(bse + NB + i, b)
            for i, b in enumerate(s1):
                g_wait(b)
                s_start(bse + NB + i, b)
            for i, b in enumerate(s0):
                s_wait(b)

                @pl.when(j < niter - 1)
                def _(i=i, b=b):
                    g_start(bse + 2 * NB + i, b)
            return carry

        lax.fori_loop(0, niter, body, 0)
        for b in s1:
            s_wait(b)

    @pl.when(c == 0)
    def _():
        run_edges(y2_hbm.at[0])

    @pl.when(c == 1)
    def _():
        run_edges(y2_hbm.at[1])
    plsc.subcore_barrier()
    pltpu.sync_copy(acc_sp.at[pl.ds(base, NPT), :], agg_out.at[c, pl.ds(base, NPT), :])


def _split_kernel_body(e_ref, s_ref, d_ref):
    s_ref[...] = e_ref[0]
    d_ref[...] = e_ref[1]


def _dense_kernel_body(H, R, x_ref, hist_ref, wg_ref, y_ref, dis_ref):
    xb = x_ref[...]
    xw = jnp.dot(xb.astype(jnp.bfloat16), wg_ref[...].astype(jnp.bfloat16),
                 preferred_element_type=F32)
    for j in range(R // 128):
        deg = jnp.sum(hist_ref[:, j * 128:(j + 1) * 128], axis=0,
                      keepdims=True) + 1.0
        dis = lax.rsqrt(deg)
        dis_ref[j] = dis
        dis_col = jnp.transpose(dis)
        y = dis_col * xw[j * 128:(j + 1) * 128]
        y_ref[0, j * 128:(j + 1) * 128] = y[:, :H]
        y_ref[1, j * 128:(j + 1) * 128] = y[:, H:]


def _final_kernel_body(R, x_ref, agg_ref, dis_ref, wg_ref, wl_ref, bg_ref, bl_ref,
                       out_ref):
    xb = x_ref[...]
    agg = jnp.concatenate([agg_ref[0], agg_ref[1]], axis=1)
    xb16 = xb.astype(jnp.bfloat16)
    xw = jnp.dot(xb16, wg_ref[...].astype(jnp.bfloat16),
                 preferred_element_type=F32)
    skip = jnp.dot(xb16, wl_ref[...].astype(jnp.bfloat16),
                   preferred_element_type=F32)
    rows = []
    for j in range(R // 128):
        sl = slice(j * 128, (j + 1) * 128)
        dis_col = jnp.transpose(dis_ref[j])
        t = agg[sl] + dis_col * xw[sl]
        rows.append(dis_col * t + skip[sl])
    sv = jnp.concatenate(rows, axis=0) + bg_ref[...] + bl_ref[...]
    u = jnp.exp(jnp.minimum(sv, 20.0))
    w = (1.0 + u) * (1.0 + u)
    out_ref[...] = sv * ((w - 1.0) / (w + 1.0))


def kernel(x, edge_index, W_gcn, b_gcn, W_lin, b_lin):
    N, D = x.shape
    E = edge_index.shape[1]
    H = D // 2
    NP = ((N + 127) // 128) * 128
    EP = ((E + 16383) // 16384) * 16384
    if NP == N:
        NP += 128
    G = NP // 128
    EA = E // 32
    EPT = EP // 16
    NB = 2

    bg2 = b_gcn.reshape(1, D)
    bl2 = b_lin.reshape(1, D)

    src_flat, dst_flat = pl.pallas_call(
        _split_kernel_body,
        out_shape=[jax.ShapeDtypeStruct((E,), jnp.int32),
                   jax.ShapeDtypeStruct((E,), jnp.int32)],
    )(edge_index)

    mesh = plsc.VectorSubcoreMesh(core_axis_name="c", subcore_axis_name="s")

    deg_kernel = functools.partial(
        pl.kernel,
        out_type=jax.ShapeDtypeStruct((32, NP), F32),
        mesh=mesh,
        compiler_params=pltpu.CompilerParams(needs_layout_passes=False),
        scratch_types=[
            pltpu.VMEM((EA,), jnp.int32),
            pltpu.VMEM((NP,), F32),
        ],
    )(functools.partial(_deg_kernel_body, NP, EA))
    hist32 = deg_kernel(dst_flat)

    R = 512
    y2, dis = pl.pallas_call(
        functools.partial(_dense_kernel_body, H, R),
        grid=((NP + R - 1) // R,),
        in_specs=[
            pl.BlockSpec((R, D), lambda i: (i, 0)),
            pl.BlockSpec((32, R), lambda i: (0, i)),
            pl.BlockSpec((D, D), lambda i: (0, 0)),
        ],
        out_specs=[
            pl.BlockSpec((2, R, H), lambda i: (0, i, 0)),
            pl.BlockSpec((R // 128, 1, 128), lambda i: (i, 0, 0)),
        ],
        out_shape=[
            jax.ShapeDtypeStruct((2, NP, H), F32),
            jax.ShapeDtypeStruct((G, 1, 128), F32),
        ],
    )(x, hist32, W_gcn)

    agg_kernel = functools.partial(
        pl.kernel,
        out_type=jax.ShapeDtypeStruct((2, NP, H), F32),
        mesh=mesh,
        compiler_params=pltpu.CompilerParams(
            needs_layout_passes=False, use_tc_tiling_on_sc=False),
        scratch_types=[
            pltpu.VMEM((EPT,), jnp.int32),
            pltpu.VMEM((EPT,), jnp.int32),
            [pltpu.VMEM((128, H), F32)] * (2 * NB),
            pltpu.VMEM_SHARED((NP, H), F32),
            [pltpu.SemaphoreType.DMA] * (2 * NB),
            [pltpu.SemaphoreType.DMA] * (2 * NB),
        ],
    )(functools.partial(_agg_kernel_body, N, NP, E, EPT, H, NB))
    agg2 = agg_kernel(y2, src_flat, dst_flat)

    out = pl.pallas_call(
        functools.partial(_final_kernel_body, R),
        grid=((N + R - 1) // R,),
        in_specs=[
            pl.BlockSpec((R, D), lambda i: (i, 0)),
            pl.BlockSpec((2, R, H), lambda i: (0, i, 0)),
            pl.BlockSpec((R // 128, 1, 128), lambda i: (i, 0, 0)),
            pl.BlockSpec((D, D), lambda i: (0, 0)),
            pl.BlockSpec((D, D), lambda i: (0, 0)),
            pl.BlockSpec((1, D), lambda i: (0, 0)),
            pl.BlockSpec((1, D), lambda i: (0, 0)),
        ],
        out_specs=pl.BlockSpec((R, D), lambda i: (i, 0)),
        out_shape=jax.ShapeDtypeStruct((N, D), F32),
    )(x, agg2, dis, W_gcn, W_lin, bg2, bl2)

    return out


# speedup vs baseline: 1.0709x; 1.0149x over previous
import functools

import jax
import jax.numpy as jnp
from jax import lax
from jax.experimental import pallas as pl
from jax.experimental.pallas import tpu as pltpu
from jax.experimental.pallas import tpu_sc as plsc

F32 = jnp.float32


def _deg_kernel_body(NP, EA, dst_hbm, hist_out, idx_v, hist_v):
    c = lax.axis_index("c")
    s = lax.axis_index("s")
    wid = s * 2 + c
    z16 = jnp.zeros((16,), F32)

    def zero_body(i, carry):
        hist_v[pl.ds(i * 16, 16)] = z16
        return carry

    lax.fori_loop(0, NP // 16, zero_body, 0)
    pltpu.sync_copy(dst_hbm.at[pl.ds(wid * EA, EA)], idx_v)
    ones16 = jnp.ones((16,), F32)
    UN = 5

    def hist_body(j, carry):
        for k in range(UN):
            idx = idx_v[pl.ds(j * (16 * UN) + k * 16, 16)]
            plsc.addupdate_scatter(hist_v, [idx], ones16)
        return carry

    lax.fori_loop(0, EA // (16 * UN), hist_body, 0)
    pltpu.sync_copy(hist_v, hist_out.at[wid])


def _agg_kernel_body(N, NP, E, EPT, H, NB, y2_hbm, src_hbm, dst_hbm, agg_out,
                     src_v, dst_v, rows, acc_sp, gsems, ssems, isems):
    c = lax.axis_index("c")
    s = lax.axis_index("s")
    NPT = NP // 16
    ER = E // 16
    z16 = jnp.zeros((16,), F32)
    zbuf = rows[0]

    icp_s = pltpu.async_copy(src_hbm.at[pl.ds(s * ER, ER)],
                             src_v.at[pl.ds(0, ER)], isems[0])
    icp_d = pltpu.async_copy(dst_hbm.at[pl.ds(s * ER, ER)],
                             dst_v.at[pl.ds(0, ER)], isems[1])

    def zbuf_body(i, carry):
        for k in range(H // 16):
            zbuf[i, pl.ds(k * 16, 16)] = z16
        return carry

    lax.fori_loop(0, 128, zbuf_body, 0)

    iota16 = lax.iota(jnp.int32, 16)
    for k in range((EPT - ER) // 16):
        padv = N + lax.rem(iota16 + (k * 16), NP - N)
        src_v[pl.ds(ER + k * 16, 16)] = padv
        dst_v[pl.ds(ER + k * 16, 16)] = padv

    base = s * NPT
    off = 0
    while off < NPT:
        sz = min(128, NPT - off)
        pltpu.sync_copy(zbuf.at[pl.ds(0, sz)], acc_sp.at[pl.ds(base + off, sz), :])
        off += sz
    plsc.subcore_barrier()
    icp_s.wait()
    icp_d.wait()

    nch = EPT // 128
    niter = nch // (2 * NB)
    s0 = tuple(range(NB))
    s1 = tuple(range(NB, 2 * NB))

    def run_edges(yref):
        def g_start(jc, b):
            pltpu.async_copy(yref.at[src_v.at[pl.ds(jc * 128, 128)]],
                             rows[b], gsems[b])

        def g_wait(b):
            pltpu.make_async_copy(yref.at[src_v.at[pl.ds(0, 128)]],
                                  rows[b], gsems[b]).wait()

        def s_start(jc, b):
            pltpu.async_copy(rows[b], acc_sp.at[dst_v.at[pl.ds(jc * 128, 128)]],
                             ssems[b], add=True)

        def s_wait(b):
            pltpu.make_async_copy(rows[b], acc_sp.at[dst_v.at[pl.ds(0, 128)]],
                                  ssems[b]).wait()

        for i, b in enumerate(s0):
            g_start(i, b)

        def body(j, carry):
            bse = 2 * j * NB
            for i, b in enumerate(s0):
                g_wait(b)
                s_start(bse + i, b)
            for i, b in enumerate(s1):
                @pl.when(j > 0)
                def _(b=b):
                    s_wait(b)
                g_start(bse + NB + i, b)
            for i, b in enumerate(s1):
                g_wait(b)
                s_start(bse + NB + i, b)
            for i, b in enumerate(s0):
                s_wait(b)

                @pl.when(j < niter - 1)
                def _(i=i, b=b):
                    g_start(bse + 2 * NB + i, b)
            return carry

        lax.fori_loop(0, niter, body, 0)
        for b in s1:
            s_wait(b)

    @pl.when(c == 0)
    def _():
        run_edges(y2_hbm.at[0])

    @pl.when(c == 1)
    def _():
        run_edges(y2_hbm.at[1])
    plsc.subcore_barrier()
    pltpu.sync_copy(acc_sp.at[pl.ds(base, NPT), :], agg_out.at[c, pl.ds(base, NPT), :])


def _split_kernel_body(e_ref, s_ref, d_ref):
    s_ref[...] = e_ref[0]
    d_ref[...] = e_ref[1]


def _dense_kernel_body(H, R, x_ref, hist_ref, wg_ref, y_ref, dis_ref):
    xb = x_ref[...]
    xw = jnp.dot(xb.astype(jnp.bfloat16), wg_ref[...].astype(jnp.bfloat16),
                 preferred_element_type=F32)
    for j in range(R // 128):
        deg = jnp.sum(hist_ref[:, j * 128:(j + 1) * 128], axis=0,
                      keepdims=True) + 1.0
        dis = lax.rsqrt(deg)
        dis_ref[j] = dis
        dis_col = jnp.transpose(dis)
        y = dis_col * xw[j * 128:(j + 1) * 128]
        y_ref[0, j * 128:(j + 1) * 128] = y[:, :H]
        y_ref[1, j * 128:(j + 1) * 128] = y[:, H:]


def _final_kernel_body(R, x_ref, agg_ref, dis_ref, wg_ref, wl_ref, bg_ref, bl_ref,
                       out_ref):
    xb = x_ref[...]
    agg = jnp.concatenate([agg_ref[0], agg_ref[1]], axis=1)
    xb16 = xb.astype(jnp.bfloat16)
    xw = jnp.dot(xb16, wg_ref[...].astype(jnp.bfloat16),
                 preferred_element_type=F32)
    skip = jnp.dot(xb16, wl_ref[...].astype(jnp.bfloat16),
                   preferred_element_type=F32)
    rows = []
    for j in range(R // 128):
        sl = slice(j * 128, (j + 1) * 128)
        dis_col = jnp.transpose(dis_ref[j])
        t = agg[sl] + dis_col * xw[sl]
        rows.append(dis_col * t + skip[sl])
    sv = jnp.concatenate(rows, axis=0) + bg_ref[...] + bl_ref[...]
    u = jnp.exp(jnp.minimum(sv, 20.0))
    w = (1.0 + u) * (1.0 + u)
    out_ref[...] = sv * ((w - 1.0) / (w + 1.0))


def kernel(x, edge_index, W_gcn, b_gcn, W_lin, b_lin):
    N, D = x.shape
    E = edge_index.shape[1]
    H = D // 2
    NP = ((N + 127) // 128) * 128
    EP = ((E + 16383) // 16384) * 16384
    if NP == N:
        NP += 128
    G = NP // 128
    EA = E // 32
    EPT = EP // 16
    NB = 2

    bg2 = b_gcn.reshape(1, D)
    bl2 = b_lin.reshape(1, D)

    src_flat, dst_flat = pl.pallas_call(
        _split_kernel_body,
        out_shape=[jax.ShapeDtypeStruct((E,), jnp.int32),
                   jax.ShapeDtypeStruct((E,), jnp.int32)],
    )(edge_index)

    mesh = plsc.VectorSubcoreMesh(core_axis_name="c", subcore_axis_name="s")

    deg_kernel = functools.partial(
        pl.kernel,
        out_type=jax.ShapeDtypeStruct((32, NP), F32),
        mesh=mesh,
        compiler_params=pltpu.CompilerParams(needs_layout_passes=False),
        scratch_types=[
            pltpu.VMEM((EA,), jnp.int32),
            pltpu.VMEM((NP,), F32),
        ],
    )(functools.partial(_deg_kernel_body, NP, EA))
    hist32 = deg_kernel(dst_flat)

    R = 512
    y2, dis = pl.pallas_call(
        functools.partial(_dense_kernel_body, H, R),
        grid=((NP + R - 1) // R,),
        in_specs=[
            pl.BlockSpec((R, D), lambda i: (i, 0)),
            pl.BlockSpec((32, R), lambda i: (0, i)),
            pl.BlockSpec((D, D), lambda i: (0, 0)),
        ],
        out_specs=[
            pl.BlockSpec((2, R, H), lambda i: (0, i, 0)),
            pl.BlockSpec((R // 128, 1, 128), lambda i: (i, 0, 0)),
        ],
        out_shape=[
            jax.ShapeDtypeStruct((2, NP, H), F32),
            jax.ShapeDtypeStruct((G, 1, 128), F32),
        ],
    )(x, hist32, W_gcn)

    agg_kernel = functools.partial(
        pl.kernel,
        out_type=jax.ShapeDtypeStruct((2, NP, H), F32),
        mesh=mesh,
        compiler_params=pltpu.CompilerParams(
            needs_layout_passes=False, use_tc_tiling_on_sc=False),
        scratch_types=[
            pltpu.VMEM((EPT,), jnp.int32),
            pltpu.VMEM((EPT,), jnp.int32),
            [pltpu.VMEM((128, H), F32)] * (2 * NB),
            pltpu.VMEM_SHARED((NP, H), F32),
            [pltpu.SemaphoreType.DMA] * (2 * NB),
            [pltpu.SemaphoreType.DMA] * (2 * NB),
            [pltpu.SemaphoreType.DMA] * 2,
        ],
    )(functools.partial(_agg_kernel_body, N, NP, E, EPT, H, NB))
    agg2 = agg_kernel(y2, src_flat, dst_flat)

    out = pl.pallas_call(
        functools.partial(_final_kernel_body, R),
        grid=((N + R - 1) // R,),
        in_specs=[
            pl.BlockSpec((R, D), lambda i: (i, 0)),
            pl.BlockSpec((2, R, H), lambda i: (0, i, 0)),
            pl.BlockSpec((R // 128, 1, 128), lambda i: (i, 0, 0)),
            pl.BlockSpec((D, D), lambda i: (0, 0)),
            pl.BlockSpec((D, D), lambda i: (0, 0)),
            pl.BlockSpec((1, D), lambda i: (0, 0)),
            pl.BlockSpec((1, D), lambda i: (0, 0)),
        ],
        out_specs=pl.BlockSpec((R, D), lambda i: (i, 0)),
        out_shape=jax.ShapeDtypeStruct((N, D), F32),
    )(x, agg2, dis, W_gcn, W_lin, bg2, bl2)

    return out
